# Initial kernel scaffold; baseline (speedup 1.0000x reference)
import jax
import jax.numpy as jnp
from jax.experimental import pallas as pl


def kernel(data):
    raise NotImplementedError("write your pallas kernel here")


# speedup vs baseline: 9.0922x; 9.0922x over previous
import functools
import math

import jax
import jax.numpy as jnp
import numpy as np
from jax import lax
from jax.experimental import pallas as pl
from jax.experimental.pallas import tpu as pltpu
from jax.experimental.pallas import tpu_sc as plsc


def _log_fact(n):
    n = n + 1
    if n < 9:
        return np.log(math.factorial(int(n)))
    return 0.5 * (np.log(2 * np.pi) - np.log(n)) + n * (
        np.log(n + 1.0 / (12 * n - 1.0 / 10 / n)) - 1
    )


def _os_threshold(k, n, pfa):
    def fun(t_os):
        return (
            _log_fact(n)
            - _log_fact(n - k)
            - np.sum(np.log(np.arange(n, n - k, -1) + t_os))
            - np.log(pfa)
        )

    t_max, t_min = 1e32, 1.0
    for _ in range(10000):
        m_n = t_max - fun(t_max) * (t_min - t_max) / (fun(t_min) - fun(t_max))
        f_m_n = fun(m_n)
        if f_m_n == 0 or np.abs(t_max - t_min) < 1e-4:
            return m_n
        if fun(t_max) * f_m_n < 0:
            t_min = m_n
        elif fun(t_min) * f_m_n < 0:
            t_max = m_n
        else:
            break
    raise ValueError("CFAR threshold did not converge")


_OS_N = 16
_K_ORDER = _OS_N * 3 / 4
_PFA = 1e-05
OS_ALPHA = float(np.sqrt(_os_threshold(_K_ORDER, _OS_N, _PFA)))
_OFFS = tuple(range(-10, -2)) + tuple(range(3, 11))
_INV_CA_N = 1.0 / 16.0

_B, _V, _R = 8, 256, 1024
_NC, _NS = 2, 16
_RCHUNK = _R // 4
_COLS = _RCHUNK + 32
_VHALF = _V // 2


def _sort4(a, b, c, d):
    mx, mn = jnp.maximum, jnp.minimum
    h1, l1 = mx(a, b), mn(a, b)
    h2, l2 = mx(c, d), mn(c, d)
    e1, t1 = mx(h1, h2), mn(h1, h2)
    t2, e4 = mx(l1, l2), mn(l1, l2)
    e2, e3 = mx(t1, t2), mn(t1, t2)
    return e1, e2, e3, e4


def _merge44_full(a, b):
    mx, mn = jnp.maximum, jnp.minimum
    a1, a2, a3, a4 = a
    b1, b2, b3, b4 = b
    c1 = mx(a1, b1)
    c2 = mx(mx(mn(a1, b1), a2), b2)
    c3 = mx(mx(mn(a1, b2), mn(a2, b1)), mx(a3, b3))
    c4 = mx(mx(mn(a1, b3), mn(a2, b2)), mx(mn(a3, b1), mx(a4, b4)))
    return c1, c2, c3, c4


def _merge44_4th(a, b):
    mx, mn = jnp.maximum, jnp.minimum
    a1, a2, a3, a4 = a
    b1, b2, b3, b4 = b
    return mx(mx(mn(a1, b3), mn(a2, b2)), mx(mn(a3, b1), mx(a4, b4)))


def _fourth_largest_16(vals):
    g = [_sort4(*vals[4 * i : 4 * i + 4]) for i in range(4)]
    return _merge44_4th(_merge44_full(g[0], g[1]), _merge44_full(g[2], g[3]))


def _cfar_body(data_hbm, out_hbm, in_slab, os_tile, out_slab):
    wid = lax.axis_index("s") * _NC + lax.axis_index("c")
    b = wid // 4
    rc = wid % 4
    r0 = rc * _RCHUNK
    start_a = (r0 + _R - 16) % _R
    start_c = (r0 + _RCHUNK) % _R

    for h in range(2):
        v0 = h * _VHALF
        pltpu.sync_copy(
            data_hbm.at[b, pl.ds(v0, _VHALF), pl.ds(start_a, 16)],
            in_slab.at[:, pl.ds(0, 16)],
        )
        pltpu.sync_copy(
            data_hbm.at[b, pl.ds(v0, _VHALF), pl.ds(r0, _RCHUNK)],
            in_slab.at[:, pl.ds(16, _RCHUNK)],
        )
        pltpu.sync_copy(
            data_hbm.at[b, pl.ds(v0, _VHALF), pl.ds(start_c, 16)],
            in_slab.at[:, pl.ds(_RCHUNK + 16, 16)],
        )

        def row_body(vi, _):
            for jb in range(_RCHUNK // 16):
                base = 16 + jb * 16
                vals = [in_slab[vi, pl.ds(base + off, 16)] for off in _OFFS]
                miu = _fourth_largest_16(vals)
                os_tile[v0 + vi, pl.ds(jb * 16, 16)] = miu * OS_ALPHA
            return 0

        lax.fori_loop(0, _VHALF, row_body, 0)

    for g in range(4):
        def ca_row(vv, _):
            v = g * 64 + vv
            for jb in range(_RCHUNK // 16):
                acc = None
                for off in _OFFS:
                    vrow = v + off
                    vrow = jnp.where(vrow < 0, vrow + _V, vrow)
                    vrow = jnp.where(vrow >= _V, vrow - _V, vrow)
                    x = os_tile[vrow, pl.ds(jb * 16, 16)]
                    acc = x if acc is None else acc + x
                out_slab[vv, pl.ds(jb * 16, 16)] = acc * _INV_CA_N
            return 0

        lax.fori_loop(0, 64, ca_row, 0)
        pltpu.sync_copy(
            out_slab, out_hbm.at[b, pl.ds(g * 64, 64), pl.ds(r0, _RCHUNK)]
        )


@jax.jit
def kernel(data):
    mesh = plsc.VectorSubcoreMesh(core_axis_name="c", subcore_axis_name="s")
    run = functools.partial(
        pl.kernel,
        mesh=mesh,
        out_type=jax.ShapeDtypeStruct((_B, _V, _R), jnp.float32),
        scratch_types=[
            pltpu.VMEM((_VHALF, _COLS), jnp.float32),
            pltpu.VMEM((_V, _RCHUNK), jnp.float32),
            pltpu.VMEM((64, _RCHUNK), jnp.float32),
        ],
        compiler_params=pltpu.CompilerParams(use_tc_tiling_on_sc=False),
    )(_cfar_body)
    return run(data)


# speedup vs baseline: 11.0045x; 1.2103x over previous
import functools
import math

import jax
import jax.numpy as jnp
import numpy as np
from jax import lax
from jax.experimental import pallas as pl
from jax.experimental.pallas import tpu as pltpu
from jax.experimental.pallas import tpu_sc as plsc


def _log_fact(n):
    n = n + 1
    if n < 9:
        return np.log(math.factorial(int(n)))
    return 0.5 * (np.log(2 * np.pi) - np.log(n)) + n * (
        np.log(n + 1.0 / (12 * n - 1.0 / 10 / n)) - 1
    )


def _os_threshold(k, n, pfa):
    def fun(t_os):
        return (
            _log_fact(n)
            - _log_fact(n - k)
            - np.sum(np.log(np.arange(n, n - k, -1) + t_os))
            - np.log(pfa)
        )

    t_max, t_min = 1e32, 1.0
    for _ in range(10000):
        m_n = t_max - fun(t_max) * (t_min - t_max) / (fun(t_min) - fun(t_max))
        f_m_n = fun(m_n)
        if f_m_n == 0 or np.abs(t_max - t_min) < 1e-4:
            return m_n
        if fun(t_max) * f_m_n < 0:
            t_min = m_n
        elif fun(t_min) * f_m_n < 0:
            t_max = m_n
        else:
            break
    raise ValueError("CFAR threshold did not converge")


_OS_N = 16
_K_ORDER = _OS_N * 3 / 4
_PFA = 1e-05
OS_ALPHA = float(np.sqrt(_os_threshold(_K_ORDER, _OS_N, _PFA)))
_OFFS = tuple(range(-10, -2)) + tuple(range(3, 11))
_INV_CA_N = 1.0 / 16.0

_B, _V, _R = 8, 256, 1024
_NC, _NS = 2, 16
_RCHUNK = _R // 4
_COLS = _RCHUNK + 32
_VHALF = _V // 2


def _sort4(a, b, c, d):
    mx, mn = jnp.maximum, jnp.minimum
    h1, l1 = mx(a, b), mn(a, b)
    h2, l2 = mx(c, d), mn(c, d)
    e1, t1 = mx(h1, h2), mn(h1, h2)
    t2, e4 = mx(l1, l2), mn(l1, l2)
    e2, e3 = mx(t1, t2), mn(t1, t2)
    return e1, e2, e3, e4


def _merge44_full(a, b):
    mx, mn = jnp.maximum, jnp.minimum
    a1, a2, a3, a4 = a
    b1, b2, b3, b4 = b
    c1 = mx(a1, b1)
    c2 = mx(mx(mn(a1, b1), a2), b2)
    c3 = mx(mx(mn(a1, b2), mn(a2, b1)), mx(a3, b3))
    c4 = mx(mx(mn(a1, b3), mn(a2, b2)), mx(mn(a3, b1), mx(a4, b4)))
    return c1, c2, c3, c4


def _merge44_4th(a, b):
    mx, mn = jnp.maximum, jnp.minimum
    a1, a2, a3, a4 = a
    b1, b2, b3, b4 = b
    return mx(mx(mn(a1, b3), mn(a2, b2)), mx(mn(a3, b1), mx(a4, b4)))


def _cfar_body(data_hbm, out_hbm, in_slab, os_tile, out_slab, m_tile):
    wid = lax.axis_index("s") * _NC + lax.axis_index("c")
    b = wid // 4
    rc = wid % 4
    r0 = rc * _RCHUNK
    start_a = (r0 + _R - 16) % _R
    start_c = (r0 + _RCHUNK) % _R

    for h in range(2):
        v0 = h * _VHALF
        pltpu.sync_copy(
            data_hbm.at[b, pl.ds(v0, _VHALF), pl.ds(start_a, 16)],
            in_slab.at[:, pl.ds(0, 16)],
        )
        pltpu.sync_copy(
            data_hbm.at[b, pl.ds(v0, _VHALF), pl.ds(r0, _RCHUNK)],
            in_slab.at[:, pl.ds(16, _RCHUNK)],
        )
        pltpu.sync_copy(
            data_hbm.at[b, pl.ds(v0, _VHALF), pl.ds(start_c, 16)],
            in_slab.at[:, pl.ds(_RCHUNK + 16, 16)],
        )

        def row_body(vi, _):
            for mb in range(17):
                x0 = 6 + mb * 16
                v8 = [in_slab[vi, pl.ds(x0 + d, 16)] for d in range(8)]
                mm = _merge44_full(_sort4(*v8[0:4]), _sort4(*v8[4:8]))
                for i in range(4):
                    m_tile[i, pl.ds(x0, 16)] = mm[i]
            for jb in range(_RCHUNK // 16):
                ml = tuple(m_tile[i, pl.ds(6 + jb * 16, 16)] for i in range(4))
                mr = tuple(m_tile[i, pl.ds(19 + jb * 16, 16)] for i in range(4))
                miu = _merge44_4th(ml, mr)
                os_tile[v0 + vi, pl.ds(jb * 16, 16)] = miu * OS_ALPHA
            return 0

        lax.fori_loop(0, _VHALF, row_body, 0)

    for g in range(4):
        vg = g * 64
        for jb in range(_RCHUNK // 16):
            cb = pl.ds(jb * 16, 16)
            acc = None
            for off in _OFFS:
                x = os_tile[(vg + off + _V) % _V, cb]
                acc = x if acc is None else acc + x

            def ca_row(vv, s):
                v = vg + vv
                out_slab[vv, cb] = s * _INV_CA_N
                s = s + os_tile[lax.rem(v + 11, _V), cb]
                s = s + os_tile[lax.rem(v + _V - 2, _V), cb]
                s = s - os_tile[lax.rem(v + _V - 10, _V), cb]
                s = s - os_tile[lax.rem(v + 3, _V), cb]
                return s

            lax.fori_loop(0, 64, ca_row, acc)
        pltpu.sync_copy(
            out_slab, out_hbm.at[b, pl.ds(g * 64, 64), pl.ds(r0, _RCHUNK)]
        )


@jax.jit
def kernel(data):
    mesh = plsc.VectorSubcoreMesh(core_axis_name="c", subcore_axis_name="s")
    run = functools.partial(
        pl.kernel,
        mesh=mesh,
        out_type=jax.ShapeDtypeStruct((_B, _V, _R), jnp.float32),
        scratch_types=[
            pltpu.VMEM((_VHALF, _COLS), jnp.float32),
            pltpu.VMEM((_V, _RCHUNK), jnp.float32),
            pltpu.VMEM((64, _RCHUNK), jnp.float32),
            pltpu.VMEM((4, _COLS), jnp.float32),
        ],
        compiler_params=pltpu.CompilerParams(use_tc_tiling_on_sc=False),
    )(_cfar_body)
    return run(data)


# speedup vs baseline: 12.0177x; 1.0921x over previous
import functools
import math

import jax
import jax.numpy as jnp
import numpy as np
from jax import lax
from jax.experimental import pallas as pl
from jax.experimental.pallas import tpu as pltpu
from jax.experimental.pallas import tpu_sc as plsc


def _log_fact(n):
    n = n + 1
    if n < 9:
        return np.log(math.factorial(int(n)))
    return 0.5 * (np.log(2 * np.pi) - np.log(n)) + n * (
        np.log(n + 1.0 / (12 * n - 1.0 / 10 / n)) - 1
    )


def _os_threshold(k, n, pfa):
    def fun(t_os):
        return (
            _log_fact(n)
            - _log_fact(n - k)
            - np.sum(np.log(np.arange(n, n - k, -1) + t_os))
            - np.log(pfa)
        )

    t_max, t_min = 1e32, 1.0
    for _ in range(10000):
        m_n = t_max - fun(t_max) * (t_min - t_max) / (fun(t_min) - fun(t_max))
        f_m_n = fun(m_n)
        if f_m_n == 0 or np.abs(t_max - t_min) < 1e-4:
            return m_n
        if fun(t_max) * f_m_n < 0:
            t_min = m_n
        elif fun(t_min) * f_m_n < 0:
            t_max = m_n
        else:
            break
    raise ValueError("CFAR threshold did not converge")


_OS_N = 16
_K_ORDER = _OS_N * 3 / 4
_PFA = 1e-05
OS_ALPHA = float(np.sqrt(_os_threshold(_K_ORDER, _OS_N, _PFA)))
_OFFS = tuple(range(-10, -2)) + tuple(range(3, 11))
_SCALE = OS_ALPHA / 16.0

_B, _V, _R = 8, 256, 1024
_NC, _NS = 2, 16
_RCHUNK = _R // 4
_COLS = _RCHUNK + 32
_VHALF = _V // 2


def _sort4(a, b, c, d):
    mx, mn = jnp.maximum, jnp.minimum
    h1, l1 = mx(a, b), mn(a, b)
    h2, l2 = mx(c, d), mn(c, d)
    e1, t1 = mx(h1, h2), mn(h1, h2)
    t2, e4 = mx(l1, l2), mn(l1, l2)
    e2, e3 = mx(t1, t2), mn(t1, t2)
    return e1, e2, e3, e4


def _merge44_full(a, b):
    mx, mn = jnp.maximum, jnp.minimum
    a1, a2, a3, a4 = a
    b1, b2, b3, b4 = b
    c1 = mx(a1, b1)
    c2 = mx(mx(mn(a1, b1), a2), b2)
    c3 = mx(mx(mn(a1, b2), mn(a2, b1)), mx(a3, b3))
    c4 = mx(mx(mn(a1, b3), mn(a2, b2)), mx(mn(a3, b1), mx(a4, b4)))
    return c1, c2, c3, c4


def _merge44_4th(a, b):
    mx, mn = jnp.maximum, jnp.minimum
    a1, a2, a3, a4 = a
    b1, b2, b3, b4 = b
    return mx(mx(mn(a1, b3), mn(a2, b2)), mx(mn(a3, b1), mx(a4, b4)))


def _os_row(in_slab, vi, m_tile, mrow, os_tile, os_row):
    for mb in range(17):
        x0 = 6 + mb * 16
        v8 = [in_slab[vi, pl.ds(x0 + d, 16)] for d in range(8)]
        mm = _merge44_full(_sort4(*v8[0:4]), _sort4(*v8[4:8]))
        for i in range(4):
            m_tile[mrow + i, pl.ds(x0, 16)] = mm[i]
    for jb in range(_RCHUNK // 16):
        ml = tuple(m_tile[mrow + i, pl.ds(6 + jb * 16, 16)] for i in range(4))
        mr = tuple(m_tile[mrow + i, pl.ds(19 + jb * 16, 16)] for i in range(4))
        os_tile[os_row, pl.ds(jb * 16, 16)] = _merge44_4th(ml, mr)


def _cfar_body(data_hbm, out_hbm, in_slab, os_tile, out_slab, m_tile):
    wid = lax.axis_index("s") * _NC + lax.axis_index("c")
    b = wid // 4
    rc = wid % 4
    r0 = rc * _RCHUNK
    start_a = (r0 + _R - 16) % _R
    start_c = (r0 + _RCHUNK) % _R

    for h in range(2):
        v0 = h * _VHALF
        pltpu.sync_copy(
            data_hbm.at[b, pl.ds(v0, _VHALF), pl.ds(start_a, 16)],
            in_slab.at[:, pl.ds(0, 16)],
        )
        pltpu.sync_copy(
            data_hbm.at[b, pl.ds(v0, _VHALF), pl.ds(r0, _RCHUNK)],
            in_slab.at[:, pl.ds(16, _RCHUNK)],
        )
        pltpu.sync_copy(
            data_hbm.at[b, pl.ds(v0, _VHALF), pl.ds(start_c, 16)],
            in_slab.at[:, pl.ds(_RCHUNK + 16, 16)],
        )

        def row_pair(vi, _):
            _os_row(in_slab, 2 * vi, m_tile, 0, os_tile, 10 + v0 + 2 * vi)
            _os_row(in_slab, 2 * vi + 1, m_tile, 4, os_tile, 11 + v0 + 2 * vi)
            return 0

        lax.fori_loop(0, _VHALF // 2, row_pair, 0)

    for hr in range(10):
        for jb in range(_RCHUNK // 16):
            os_tile[hr, pl.ds(jb * 16, 16)] = os_tile[256 + hr, pl.ds(jb * 16, 16)]
    for hr in range(11):
        for jb in range(_RCHUNK // 16):
            os_tile[266 + hr, pl.ds(jb * 16, 16)] = os_tile[10 + hr, pl.ds(jb * 16, 16)]

    for g in range(4):
        vg = g * 64
        for jb in range(_RCHUNK // 16):
            cb = pl.ds(jb * 16, 16)
            acc = None
            for off in _OFFS:
                x = os_tile[vg + 10 + off, cb]
                acc = x if acc is None else acc + x

            def ca_rows(vv, s):
                v = vg + 4 * vv
                for u in range(4):
                    out_slab[4 * vv + u, cb] = s * _SCALE
                    d = (os_tile[v + u + 21, cb] + os_tile[v + u + 8, cb]) - (
                        os_tile[v + u, cb] + os_tile[v + u + 13, cb]
                    )
                    s = s + d
                return s

            lax.fori_loop(0, 16, ca_rows, acc)
        pltpu.sync_copy(
            out_slab, out_hbm.at[b, pl.ds(g * 64, 64), pl.ds(r0, _RCHUNK)]
        )


@jax.jit
def kernel(data):
    mesh = plsc.VectorSubcoreMesh(core_axis_name="c", subcore_axis_name="s")
    run = functools.partial(
        pl.kernel,
        mesh=mesh,
        out_type=jax.ShapeDtypeStruct((_B, _V, _R), jnp.float32),
        scratch_types=[
            pltpu.VMEM((_VHALF, _COLS), jnp.float32),
            pltpu.VMEM((_V + 21, _RCHUNK), jnp.float32),
            pltpu.VMEM((64, _RCHUNK), jnp.float32),
            pltpu.VMEM((8, _COLS), jnp.float32),
        ],
        compiler_params=pltpu.CompilerParams(use_tc_tiling_on_sc=False),
    )(_cfar_body)
    return run(data)
